# baseline (speedup 1.0000x reference)
import functools

import jax
import jax.numpy as jnp
from jax.experimental import pallas as pl
from jax.experimental.pallas import tpu as pltpu
from jax.experimental.pallas import tpu_sc as plsc

D_MODEL = 1024
HIDDEN = 4096
E = 8
N_TOK = 2048
TOP_K = 2

A_TOT = N_TOK * TOP_K
B_ROW = 256
NB = A_TOT // B_ROW + E
R_PAD = NB * B_ROW
H_BLK = 1024
NH = HIDDEN // H_BLK

NW = 32
T_PER_W = N_TOK // NW


def _gelu_tanh(u):
    return jax.nn.gelu(u, approximate=True)


def _router_body(x_ref, wr_ref, br_ref,
                 dest_ref, g_ref, brow_ref, bexp_ref, bval_ref,
                 oh_scr, c_scr):
    logits = jnp.dot(x_ref[...], wr_ref[...],
                     preferred_element_type=jnp.float32) + br_ref[...][None, :]
    idxv = jax.lax.broadcasted_iota(jnp.int32, (1, E), 1)
    m1 = jnp.max(logits, axis=1, keepdims=True)
    i1 = jnp.min(jnp.where(logits == m1, idxv, E), axis=1, keepdims=True)
    masked = jnp.where(idxv == i1, jnp.float32(-1e30), logits)
    m2 = jnp.max(masked, axis=1, keepdims=True)
    i2 = jnp.min(jnp.where(masked == m2, idxv, E), axis=1, keepdims=True)
    z = jnp.exp(m2 - m1)
    g1 = 1.0 / (1.0 + z)
    g2 = z / (1.0 + z)
    g_ref[...] = jnp.concatenate([g1, g2], axis=0)

    oh_scr[...] = jnp.concatenate(
        [(idxv == i1).astype(jnp.float32), (idxv == i2).astype(jnp.float32)],
        axis=0)

    riota = jax.lax.broadcasted_iota(jnp.int32, (B_ROW, B_ROW), 0)
    ciota = jax.lax.broadcasted_iota(jnp.int32, (B_ROW, B_ROW), 1)
    ltri = (riota >= ciota).astype(jnp.float32)

    def chunk(i, base):
        off = pl.multiple_of(i * B_ROW, B_ROW)
        oh = oh_scr[pl.ds(off, B_ROW), :]
        c = jnp.dot(ltri, oh, preferred_element_type=jnp.float32) + base
        c_scr[pl.ds(off, B_ROW), :] = c
        return c[B_ROW - 1:B_ROW, :]

    counts = jax.lax.fori_loop(0, A_TOT // B_ROW, chunk,
                               jnp.zeros((1, E), jnp.float32))

    nb = jnp.floor((counts + (B_ROW - 1.0)) * (1.0 / B_ROW))
    r8 = jax.lax.broadcasted_iota(jnp.int32, (E, E), 0)
    c8 = jax.lax.broadcasted_iota(jnp.int32, (E, E), 1)
    utri8 = (r8 <= c8).astype(jnp.float32)
    bo = jnp.dot(nb, utri8, preferred_element_type=jnp.float32)
    po = (bo - nb) * float(B_ROW)

    oh = oh_scr[...]
    dest_f = jnp.sum(oh * (c_scr[...] - 1.0 + po), axis=1, keepdims=True)
    dest_ref[...] = dest_f.astype(jnp.int32)

    biota = jax.lax.broadcasted_iota(jnp.int32, (1, NB), 1).astype(jnp.float32)
    nblocks = bo[0:1, E - 1:E]
    bclamp = jnp.minimum(biota, nblocks - 1.0)
    eacc = jnp.zeros((1, NB), jnp.float32)
    for e in range(E):
        eacc = eacc + (bclamp >= bo[0:1, e:e + 1]).astype(jnp.float32)
    brow_ref[...] = bclamp.astype(jnp.int32)
    bexp_ref[...] = eacc.astype(jnp.int32)
    bval_ref[...] = (biota < nblocks).astype(jnp.int32)


def _router_call(x, Wr, br):
    return pl.pallas_call(
        _router_body,
        out_shape=[
            jax.ShapeDtypeStruct((A_TOT, 1), jnp.int32),
            jax.ShapeDtypeStruct((A_TOT, 1), jnp.float32),
            jax.ShapeDtypeStruct((1, NB), jnp.int32),
            jax.ShapeDtypeStruct((1, NB), jnp.int32),
            jax.ShapeDtypeStruct((1, NB), jnp.int32),
        ],
        scratch_shapes=[
            pltpu.VMEM((A_TOT, E), jnp.float32),
            pltpu.VMEM((A_TOT, E), jnp.float32),
        ],
    )(x, Wr, br)


def _gemm_body(brow_s, bexp_s, bval_s,
               x_ref, w1_ref, b1_ref, w2_ref, b2_ref, y_ref, acc):
    h = pl.program_id(0)
    b = pl.program_id(1)

    @pl.when(bval_s[b] == 1)
    def _():
        hp = jnp.dot(x_ref[...].astype(jnp.bfloat16),
                     w1_ref[0].astype(jnp.bfloat16),
                     preferred_element_type=jnp.float32) + b1_ref[0]
        hp = _gelu_tanh(hp)
        contrib = jnp.dot(hp.astype(jnp.bfloat16),
                          w2_ref[0].astype(jnp.bfloat16),
                          preferred_element_type=jnp.float32)
        row = pl.multiple_of(b * B_ROW, B_ROW)
        sl = pl.ds(row, B_ROW)

        @pl.when(h == 0)
        def _():
            acc[sl, :] = contrib + b2_ref[0]

        @pl.when(h > 0)
        def _():
            acc[sl, :] += contrib

        @pl.when(h == NH - 1)
        def _():
            y_ref[...] = acc[sl, :]


def _gemm_call(brow, bexp, bval, xs, W1, b1r, W2, b2r):
    grid_spec = pltpu.PrefetchScalarGridSpec(
        num_scalar_prefetch=3,
        grid=(NH, NB),
        in_specs=[
            pl.BlockSpec((B_ROW, D_MODEL),
                         lambda h, b, br_s, be_s, bv_s: (br_s[b], 0)),
            pl.BlockSpec((1, D_MODEL, H_BLK),
                         lambda h, b, br_s, be_s, bv_s: (be_s[b], 0, h)),
            pl.BlockSpec((1, 1, H_BLK),
                         lambda h, b, br_s, be_s, bv_s: (be_s[b], 0, h)),
            pl.BlockSpec((1, H_BLK, D_MODEL),
                         lambda h, b, br_s, be_s, bv_s: (be_s[b], h, 0)),
            pl.BlockSpec((1, 1, D_MODEL),
                         lambda h, b, br_s, be_s, bv_s: (be_s[b], 0, 0)),
        ],
        out_specs=pl.BlockSpec(
            (B_ROW, D_MODEL),
            lambda h, b, br_s, be_s, bv_s: (
                jnp.where(h == NH - 1, br_s[b], 0), 0)),
        scratch_shapes=[pltpu.VMEM((R_PAD, D_MODEL), jnp.float32)],
    )
    return pl.pallas_call(
        _gemm_body,
        grid_spec=grid_spec,
        out_shape=jax.ShapeDtypeStruct((R_PAD, D_MODEL), jnp.float32),
        compiler_params=pltpu.CompilerParams(
            dimension_semantics=("arbitrary", "arbitrary"),
        ),
    )(brow, bexp, bval, xs, W1, b1r, W2, b2r)


_C_CHUNK = 32


@functools.lru_cache(maxsize=None)
def _sc_kernels():
    mesh = plsc.VectorSubcoreMesh(core_axis_name="c", subcore_axis_name="s")

    @functools.partial(
        pl.kernel,
        mesh=mesh,
        out_type=jax.ShapeDtypeStruct((R_PAD, D_MODEL), jnp.float32),
        scratch_types=[
            pltpu.VMEM((T_PER_W, D_MODEL), jnp.float32),
            pltpu.VMEM((TOP_K, T_PER_W), jnp.int32),
            pltpu.SemaphoreType.DMA,
            pltpu.SemaphoreType.DMA,
        ],
    )
    def sc_scatter(x_hbm, dflat_hbm, xs_hbm, rows_v, idx_v, sem0, sem1):
        wid = jax.lax.axis_index("s") * 2 + jax.lax.axis_index("c")
        base = wid * T_PER_W
        pltpu.sync_copy(x_hbm.at[pl.ds(base, T_PER_W)], rows_v)
        pltpu.sync_copy(
            dflat_hbm.at[pl.ds(base, T_PER_W)], idx_v.at[0])
        pltpu.sync_copy(
            dflat_hbm.at[pl.ds(N_TOK + base, T_PER_W)], idx_v.at[1])
        cp0 = pltpu.async_copy(rows_v, xs_hbm.at[idx_v.at[0]], sem0)
        cp1 = pltpu.async_copy(rows_v, xs_hbm.at[idx_v.at[1]], sem1)
        cp0.wait()
        cp1.wait()

    @functools.partial(
        pl.kernel,
        mesh=mesh,
        out_type=[
            jax.ShapeDtypeStruct((N_TOK, D_MODEL), jnp.float32),
            jax.ShapeDtypeStruct((N_TOK, D_MODEL), jnp.float32),
        ],
        scratch_types=[
            pltpu.VMEM((TOP_K, T_PER_W), jnp.int32),
            pltpu.VMEM((_C_CHUNK, D_MODEL), jnp.float32),
            pltpu.VMEM((_C_CHUNK, D_MODEL), jnp.float32),
            pltpu.SemaphoreType.DMA,
            pltpu.SemaphoreType.DMA,
        ],
    )
    def sc_combine(y_hbm, dflat_hbm, ya_hbm, yb_hbm,
                   idx_v, buf_a, buf_b, sem_a, sem_b):
        wid = jax.lax.axis_index("s") * 2 + jax.lax.axis_index("c")
        base = wid * T_PER_W
        pltpu.sync_copy(
            dflat_hbm.at[pl.ds(base, T_PER_W)], idx_v.at[0])
        pltpu.sync_copy(
            dflat_hbm.at[pl.ds(N_TOK + base, T_PER_W)], idx_v.at[1])
        for cc in range(T_PER_W // _C_CHUNK):
            off = cc * _C_CHUNK
            cpa = pltpu.async_copy(
                y_hbm.at[idx_v.at[0, pl.ds(off, _C_CHUNK)]], buf_a, sem_a)
            cpb = pltpu.async_copy(
                y_hbm.at[idx_v.at[1, pl.ds(off, _C_CHUNK)]], buf_b, sem_b)
            cpa.wait()
            cpb.wait()
            pltpu.sync_copy(buf_a, ya_hbm.at[pl.ds(base + off, _C_CHUNK)])
            pltpu.sync_copy(buf_b, yb_hbm.at[pl.ds(base + off, _C_CHUNK)])

    return sc_scatter, sc_combine


def _mix_body(ya_ref, yb_ref, g0_ref, g1_ref, o_ref):
    o_ref[...] = ya_ref[...] * g0_ref[...] + yb_ref[...] * g1_ref[...]


def _mix_call(ya, yb, g0, g1):
    blk = 256
    return pl.pallas_call(
        _mix_body,
        grid=(N_TOK // blk,),
        in_specs=[
            pl.BlockSpec((blk, D_MODEL), lambda t: (t, 0)),
            pl.BlockSpec((blk, D_MODEL), lambda t: (t, 0)),
            pl.BlockSpec((blk, 1), lambda t: (t, 0)),
            pl.BlockSpec((blk, 1), lambda t: (t, 0)),
        ],
        out_specs=pl.BlockSpec((blk, D_MODEL), lambda t: (t, 0)),
        out_shape=jax.ShapeDtypeStruct((N_TOK, D_MODEL), jnp.float32),
    )(ya, yb, g0, g1)


def kernel(x, Wr, br, W1, b1, W2, b2):
    dest, gall, brow, bexp, bval = _router_call(x, Wr, br)
    dflat = dest.reshape(A_TOT)
    sc_scatter, sc_combine = _sc_kernels()
    xs = sc_scatter(x, dflat)
    y = _gemm_call(brow.reshape(NB), bexp.reshape(NB), bval.reshape(NB),
                   xs, W1, b1.reshape(E, 1, HIDDEN), W2,
                   b2.reshape(E, 1, D_MODEL))
    ya, yb = sc_combine(y, dflat)
    return _mix_call(ya, yb, gall[:N_TOK], gall[N_TOK:])


# speedup vs baseline: 1.0337x; 1.0337x over previous
import functools

import jax
import jax.numpy as jnp
from jax.experimental import pallas as pl
from jax.experimental.pallas import tpu as pltpu
from jax.experimental.pallas import tpu_sc as plsc

D_MODEL = 1024
HIDDEN = 4096
E = 8
N_TOK = 2048
TOP_K = 2

A_TOT = N_TOK * TOP_K
B_ROW = 256
NB = A_TOT // B_ROW + E
R_PAD = NB * B_ROW
H_BLK = 1024
NH = HIDDEN // H_BLK

NW = 32
T_PER_W = N_TOK // NW


def _gelu_tanh(u):
    return jax.nn.gelu(u, approximate=True)


def _router_body(x_ref, wr_ref, br_ref,
                 dest_ref, g_ref, brow_ref, bexp_ref, bval_ref,
                 oh_scr, c_scr):
    logits = jnp.dot(x_ref[...], wr_ref[...],
                     preferred_element_type=jnp.float32) + br_ref[...][None, :]
    idxv = jax.lax.broadcasted_iota(jnp.int32, (1, E), 1)
    m1 = jnp.max(logits, axis=1, keepdims=True)
    i1 = jnp.min(jnp.where(logits == m1, idxv, E), axis=1, keepdims=True)
    masked = jnp.where(idxv == i1, jnp.float32(-1e30), logits)
    m2 = jnp.max(masked, axis=1, keepdims=True)
    i2 = jnp.min(jnp.where(masked == m2, idxv, E), axis=1, keepdims=True)
    z = jnp.exp(m2 - m1)
    g1 = 1.0 / (1.0 + z)
    g2 = z / (1.0 + z)
    g_ref[...] = jnp.concatenate([g1, g2], axis=0)

    oh_scr[...] = jnp.concatenate(
        [(idxv == i1).astype(jnp.float32), (idxv == i2).astype(jnp.float32)],
        axis=0)

    scn = 1024
    riota = jax.lax.broadcasted_iota(jnp.int32, (scn, scn), 0)
    ciota = jax.lax.broadcasted_iota(jnp.int32, (scn, scn), 1)
    ltri = (riota >= ciota).astype(jnp.float32)

    def chunk(i, base):
        off = pl.multiple_of(i * scn, scn)
        oh = oh_scr[pl.ds(off, scn), :]
        c = jnp.dot(ltri, oh, preferred_element_type=jnp.float32) + base
        c_scr[pl.ds(off, scn), :] = c
        return c[scn - 1:scn, :]

    counts = jax.lax.fori_loop(0, A_TOT // scn, chunk,
                               jnp.zeros((1, E), jnp.float32))

    nb = jnp.floor((counts + (B_ROW - 1.0)) * (1.0 / B_ROW))
    r8 = jax.lax.broadcasted_iota(jnp.int32, (E, E), 0)
    c8 = jax.lax.broadcasted_iota(jnp.int32, (E, E), 1)
    utri8 = (r8 <= c8).astype(jnp.float32)
    bo = jnp.dot(nb, utri8, preferred_element_type=jnp.float32)
    po = (bo - nb) * float(B_ROW)

    oh = oh_scr[...]
    dest_f = jnp.sum(oh * (c_scr[...] - 1.0 + po), axis=1, keepdims=True)
    dest_ref[...] = dest_f.astype(jnp.int32)

    biota = jax.lax.broadcasted_iota(jnp.int32, (1, NB), 1).astype(jnp.float32)
    nblocks = bo[0:1, E - 1:E]
    bclamp = jnp.minimum(biota, nblocks - 1.0)
    eacc = jnp.zeros((1, NB), jnp.float32)
    for e in range(E):
        eacc = eacc + (bclamp >= bo[0:1, e:e + 1]).astype(jnp.float32)
    brow_ref[...] = bclamp.astype(jnp.int32)
    bexp_ref[...] = eacc.astype(jnp.int32)
    bval_ref[...] = (biota < nblocks).astype(jnp.int32)


def _router_call(x, Wr, br):
    return pl.pallas_call(
        _router_body,
        out_shape=[
            jax.ShapeDtypeStruct((A_TOT, 1), jnp.int32),
            jax.ShapeDtypeStruct((A_TOT, 1), jnp.float32),
            jax.ShapeDtypeStruct((1, NB), jnp.int32),
            jax.ShapeDtypeStruct((1, NB), jnp.int32),
            jax.ShapeDtypeStruct((1, NB), jnp.int32),
        ],
        scratch_shapes=[
            pltpu.VMEM((A_TOT, E), jnp.float32),
            pltpu.VMEM((A_TOT, E), jnp.float32),
        ],
    )(x, Wr, br)


def _gemm_body(brow_s, bexp_s, bval_s,
               x_ref, w1_ref, b1_ref, w2_ref, b2_ref, y_ref, acc):
    h = pl.program_id(0)
    b = pl.program_id(1)

    @pl.when(bval_s[b] == 1)
    def _():
        hp = jnp.dot(x_ref[...], w1_ref[0],
                     preferred_element_type=jnp.float32) + b1_ref[0]
        hp = _gelu_tanh(hp)
        contrib = jnp.dot(hp, w2_ref[0], preferred_element_type=jnp.float32)
        row = pl.multiple_of(b * B_ROW, B_ROW)
        sl = pl.ds(row, B_ROW)

        @pl.when(h == 0)
        def _():
            acc[sl, :] = contrib + b2_ref[0]

        @pl.when(h > 0)
        def _():
            acc[sl, :] += contrib

        @pl.when(h == NH - 1)
        def _():
            y_ref[...] = acc[sl, :]


def _gemm_call(brow, bexp, bval, xs, W1, b1r, W2, b2r):
    grid_spec = pltpu.PrefetchScalarGridSpec(
        num_scalar_prefetch=3,
        grid=(NH, NB),
        in_specs=[
            pl.BlockSpec((B_ROW, D_MODEL),
                         lambda h, b, br_s, be_s, bv_s: (br_s[b], 0)),
            pl.BlockSpec((1, D_MODEL, H_BLK),
                         lambda h, b, br_s, be_s, bv_s: (be_s[b], 0, h)),
            pl.BlockSpec((1, 1, H_BLK),
                         lambda h, b, br_s, be_s, bv_s: (be_s[b], 0, h)),
            pl.BlockSpec((1, H_BLK, D_MODEL),
                         lambda h, b, br_s, be_s, bv_s: (be_s[b], h, 0)),
            pl.BlockSpec((1, 1, D_MODEL),
                         lambda h, b, br_s, be_s, bv_s: (be_s[b], 0, 0)),
        ],
        out_specs=pl.BlockSpec(
            (B_ROW, D_MODEL),
            lambda h, b, br_s, be_s, bv_s: (
                jnp.where(h == NH - 1, br_s[b], 0), 0)),
        scratch_shapes=[pltpu.VMEM((R_PAD, D_MODEL), jnp.float32)],
    )
    return pl.pallas_call(
        _gemm_body,
        grid_spec=grid_spec,
        out_shape=jax.ShapeDtypeStruct((R_PAD, D_MODEL), jnp.float32),
        compiler_params=pltpu.CompilerParams(
            dimension_semantics=("arbitrary", "arbitrary"),
        ),
    )(brow, bexp, bval, xs, W1, b1r, W2, b2r)


_C_CHUNK = 32


@functools.lru_cache(maxsize=None)
def _sc_kernels():
    mesh = plsc.VectorSubcoreMesh(core_axis_name="c", subcore_axis_name="s")

    @functools.partial(
        pl.kernel,
        mesh=mesh,
        out_type=jax.ShapeDtypeStruct((R_PAD, D_MODEL), jnp.float32),
        scratch_types=[
            pltpu.VMEM((T_PER_W, D_MODEL), jnp.float32),
            pltpu.VMEM((TOP_K, T_PER_W), jnp.int32),
            pltpu.SemaphoreType.DMA,
            pltpu.SemaphoreType.DMA,
        ],
    )
    def sc_scatter(x_hbm, dflat_hbm, xs_hbm, rows_v, idx_v, sem0, sem1):
        wid = jax.lax.axis_index("s") * 2 + jax.lax.axis_index("c")
        base = wid * T_PER_W
        pltpu.sync_copy(x_hbm.at[pl.ds(base, T_PER_W)], rows_v)
        pltpu.sync_copy(
            dflat_hbm.at[pl.ds(base, T_PER_W)], idx_v.at[0])
        pltpu.sync_copy(
            dflat_hbm.at[pl.ds(N_TOK + base, T_PER_W)], idx_v.at[1])
        cp0 = pltpu.async_copy(rows_v, xs_hbm.at[idx_v.at[0]], sem0)
        cp1 = pltpu.async_copy(rows_v, xs_hbm.at[idx_v.at[1]], sem1)
        cp0.wait()
        cp1.wait()

    @functools.partial(
        pl.kernel,
        mesh=mesh,
        out_type=[
            jax.ShapeDtypeStruct((N_TOK, D_MODEL), jnp.float32),
            jax.ShapeDtypeStruct((N_TOK, D_MODEL), jnp.float32),
        ],
        scratch_types=[
            pltpu.VMEM((TOP_K, T_PER_W), jnp.int32),
            pltpu.VMEM((_C_CHUNK, D_MODEL), jnp.float32),
            pltpu.VMEM((_C_CHUNK, D_MODEL), jnp.float32),
            pltpu.SemaphoreType.DMA,
            pltpu.SemaphoreType.DMA,
        ],
    )
    def sc_combine(y_hbm, dflat_hbm, ya_hbm, yb_hbm,
                   idx_v, buf_a, buf_b, sem_a, sem_b):
        wid = jax.lax.axis_index("s") * 2 + jax.lax.axis_index("c")
        base = wid * T_PER_W
        pltpu.sync_copy(
            dflat_hbm.at[pl.ds(base, T_PER_W)], idx_v.at[0])
        pltpu.sync_copy(
            dflat_hbm.at[pl.ds(N_TOK + base, T_PER_W)], idx_v.at[1])
        for cc in range(T_PER_W // _C_CHUNK):
            off = cc * _C_CHUNK
            cpa = pltpu.async_copy(
                y_hbm.at[idx_v.at[0, pl.ds(off, _C_CHUNK)]], buf_a, sem_a)
            cpb = pltpu.async_copy(
                y_hbm.at[idx_v.at[1, pl.ds(off, _C_CHUNK)]], buf_b, sem_b)
            cpa.wait()
            cpb.wait()
            pltpu.sync_copy(buf_a, ya_hbm.at[pl.ds(base + off, _C_CHUNK)])
            pltpu.sync_copy(buf_b, yb_hbm.at[pl.ds(base + off, _C_CHUNK)])

    return sc_scatter, sc_combine


def _mix_body(ya_ref, yb_ref, g0_ref, g1_ref, o_ref):
    o_ref[...] = ya_ref[...] * g0_ref[...] + yb_ref[...] * g1_ref[...]


def _mix_call(ya, yb, g0, g1):
    blk = 256
    return pl.pallas_call(
        _mix_body,
        grid=(N_TOK // blk,),
        in_specs=[
            pl.BlockSpec((blk, D_MODEL), lambda t: (t, 0)),
            pl.BlockSpec((blk, D_MODEL), lambda t: (t, 0)),
            pl.BlockSpec((blk, 1), lambda t: (t, 0)),
            pl.BlockSpec((blk, 1), lambda t: (t, 0)),
        ],
        out_specs=pl.BlockSpec((blk, D_MODEL), lambda t: (t, 0)),
        out_shape=jax.ShapeDtypeStruct((N_TOK, D_MODEL), jnp.float32),
    )(ya, yb, g0, g1)


def kernel(x, Wr, br, W1, b1, W2, b2):
    dest, gall, brow, bexp, bval = _router_call(x, Wr, br)
    dflat = dest.reshape(A_TOT)
    sc_scatter, sc_combine = _sc_kernels()
    xs = sc_scatter(x, dflat)
    y = _gemm_call(brow.reshape(NB), bexp.reshape(NB), bval.reshape(NB),
                   xs, W1, b1.reshape(E, 1, HIDDEN), W2,
                   b2.reshape(E, 1, D_MODEL))
    ya, yb = sc_combine(y, dflat)
    return _mix_call(ya, yb, gall[:N_TOK], gall[N_TOK:])
